# baseline (speedup 1.0000x reference)
import functools

import jax
import jax.numpy as jnp
from jax import lax
from jax.experimental import pallas as pl
from jax.experimental.pallas import tpu as pltpu
from jax.experimental.pallas import tpu_sc as plsc

N = 10000
E = 320000
F = 128

NC = 2
NS = 16
NW = NC * NS
EPW = E // NW
K = 128
EPWP = 10240
CH = EPWP // K
PASSES = 2
CHP = CH // PASSES
ROWS = 624
TAIL = N - NS * ROWS
LANES = 16


def _mm(h, W, b, *, relu=False):
    def body(h_ref, w_ref, b_ref, o_ref):
        h_blk = h_ref[...]
        if relu:
            h_blk = jnp.maximum(h_blk, 0.0)
        o_ref[...] = (
            jnp.dot(h_blk, w_ref[...], preferred_element_type=jnp.float32)
            + b_ref[...]
        )

    return pl.pallas_call(
        body,
        grid=(10,),
        in_specs=[
            pl.BlockSpec((N // 10, F), lambda i: (i, 0)),
            pl.BlockSpec((F, F), lambda i: (0, 0)),
            pl.BlockSpec((1, F), lambda i: (0, 0)),
        ],
        out_specs=pl.BlockSpec((N // 10, F), lambda i: (i, 0)),
        out_shape=jax.ShapeDtypeStruct((N, F), jnp.float32),
    )(h, W, b)


def _mm_fused(p, W, b, *, relu):
    def body(p_ref, w_ref, b_ref, o_ref):
        h_blk = p_ref[0] + p_ref[1]
        if relu:
            h_blk = jnp.maximum(h_blk, 0.0)
        o_ref[...] = (
            jnp.dot(h_blk, w_ref[...], preferred_element_type=jnp.float32)
            + b_ref[...]
        )

    return pl.pallas_call(
        body,
        grid=(10,),
        in_specs=[
            pl.BlockSpec((2, N // 10, F), lambda i: (0, i, 0)),
            pl.BlockSpec((F, F), lambda i: (0, 0)),
            pl.BlockSpec((1, F), lambda i: (0, 0)),
        ],
        out_specs=pl.BlockSpec((N // 10, F), lambda i: (i, 0)),
        out_shape=jax.ShapeDtypeStruct((N, F), jnp.float32),
    )(p, W, b)


def _add2(p):
    def body(p_ref, o_ref):
        o_ref[...] = p_ref[0] + p_ref[1]

    return pl.pallas_call(
        body,
        grid=(10,),
        in_specs=[pl.BlockSpec((2, N // 10, F), lambda i: (0, i, 0))],
        out_specs=pl.BlockSpec((N // 10, F), lambda i: (i, 0)),
        out_shape=jax.ShapeDtypeStruct((N, F), jnp.float32),
    )(p)


_MESH = plsc.VectorSubcoreMesh(
    core_axis_name="c", subcore_axis_name="s", num_cores=NC, num_subcores=NS
)


@functools.partial(
    pl.kernel,
    out_type=jax.ShapeDtypeStruct((2, N, F), jnp.float32),
    mesh=_MESH,
    scratch_types=[
        pltpu.VMEM((CHP, K), jnp.int32),
        pltpu.VMEM((CHP, K), jnp.int32),
        pltpu.VMEM((CHP, K), jnp.float32),
        pltpu.VMEM((2, K, F), jnp.float32),
        pltpu.VMEM_SHARED((N, F), jnp.float32),
        pltpu.SemaphoreType.DMA((2,)),
    ],
)
def _prop_kernel(u_hbm, src_hbm, dst_hbm, w_hbm, out_hbm,
                 src_v, dst_v, w_v, buf, acc, gsems):
    c = lax.axis_index("c")
    s = lax.axis_index("s")
    wid = s * NC + c

    zero = jnp.zeros((LANES,), jnp.float32)

    def zrow(i, carry):
        for q in range(F // LANES):
            buf[0, i, pl.ds(q * LANES, LANES)] = zero
        return carry

    lax.fori_loop(0, K, zrow, 0)
    for off in range(0, ROWS, K):
        zn = min(K, ROWS - off)
        pltpu.sync_copy(buf.at[0, pl.ds(0, zn)],
                        acc.at[pl.ds(s * ROWS + off, zn)])

    @pl.when(s == NS - 1)
    def _zero_tail():
        pltpu.sync_copy(buf.at[0, pl.ds(0, TAIL)],
                        acc.at[pl.ds(NS * ROWS, TAIL)])

    plsc.subcore_barrier()

    def scale(bufb, j):
        def group(g, carry2):
            w16 = w_v[j, pl.ds(g * LANES, LANES)]
            base = g * LANES
            for t in range(LANES):
                wsp = lax.gather(
                    w16, jnp.full((LANES, 1), t, jnp.int32),
                    lax.GatherDimensionNumbers(offset_dims=(),
                                               collapsed_slice_dims=(0,),
                                               start_index_map=(0,)),
                    (1,), mode=lax.GatherScatterMode.PROMISE_IN_BOUNDS)
                for q in range(F // LANES):
                    sl = pl.ds(q * LANES, LANES)
                    bufb[base + t, sl] = bufb[base + t, sl] * wsp
            return carry2

        lax.fori_loop(0, K // LANES, group, 0)

    for p in range(PASSES):
        pltpu.sync_copy(src_hbm.at[wid, p], src_v)
        pltpu.sync_copy(dst_hbm.at[wid, p], dst_v)
        pltpu.sync_copy(w_hbm.at[wid, p], w_v)
        pltpu.async_copy(u_hbm.at[src_v.at[0]], buf.at[0], gsems.at[0])

        def step_fn(j, carry):
            b = lax.rem(j, 2)
            bn = 1 - b
            jn = jnp.minimum(j + 1, CHP - 1)
            pltpu.async_copy(u_hbm.at[src_v.at[jn]], buf.at[bn],
                             gsems.at[bn])
            pltpu.make_async_copy(u_hbm.at[src_v.at[j]], buf.at[b],
                                  gsems.at[b]).wait()
            scale(buf.at[b], j)
            pltpu.sync_copy(buf.at[b], acc.at[dst_v.at[j]], add=True)
            return carry

        lax.fori_loop(0, CHP, step_fn, 0)
        pltpu.make_async_copy(u_hbm.at[src_v.at[CHP - 1]], buf.at[0],
                              gsems.at[0]).wait()

    plsc.subcore_barrier()
    pltpu.sync_copy(acc.at[pl.ds(s * ROWS, ROWS)],
                    out_hbm.at[c, pl.ds(s * ROWS, ROWS)])

    @pl.when(s == NS - 1)
    def _copy_tail():
        pltpu.sync_copy(acc.at[pl.ds(NS * ROWS, TAIL)],
                        out_hbm.at[c, pl.ds(NS * ROWS, TAIL)])


def kernel(x, edge_index, edge_weight, W1, b1, W2, b2, W3, b3):
    pad = ((0, 0), (0, EPWP - EPW))
    src = jnp.pad(edge_index[0].reshape(NW, EPW), pad).reshape(NW, PASSES, CHP, K)
    dst = jnp.pad(edge_index[1].reshape(NW, EPW), pad).reshape(NW, PASSES, CHP, K)
    w = jnp.pad(edge_weight.reshape(NW, EPW), pad).reshape(NW, PASSES, CHP, K)

    u = _mm(x, W1, b1.reshape(1, F))
    p = _prop_kernel(u, src, dst, w)
    u = _mm_fused(p, W2, b2.reshape(1, F), relu=True)
    p = _prop_kernel(u, src, dst, w)
    u = _mm_fused(p, W3, b3.reshape(1, F), relu=True)
    p = _prop_kernel(u, src, dst, w)
    return _add2(p)


# speedup vs baseline: 1.5009x; 1.5009x over previous
import functools

import jax
import jax.numpy as jnp
from jax import lax
from jax.experimental import pallas as pl
from jax.experimental.pallas import tpu as pltpu
from jax.experimental.pallas import tpu_sc as plsc

N = 10000
E = 320000
F = 128

NC = 2
NS = 16
NW = NC * NS
EPW = E // NW
K = 128
EPWP = 10240
CH = EPWP // K
PASSES = 2
CHP = CH // PASSES
ROWS = 624
TAIL = N - NS * ROWS
LANES = 16


def _mm(h, W, b, *, relu=False):
    def body(h_ref, w_ref, b_ref, o_ref):
        h_blk = h_ref[...]
        if relu:
            h_blk = jnp.maximum(h_blk, 0.0)
        o_ref[...] = (
            jnp.dot(h_blk, w_ref[...], preferred_element_type=jnp.float32)
            + b_ref[...]
        )

    return pl.pallas_call(
        body,
        grid=(10,),
        in_specs=[
            pl.BlockSpec((N // 10, F), lambda i: (i, 0)),
            pl.BlockSpec((F, F), lambda i: (0, 0)),
            pl.BlockSpec((1, F), lambda i: (0, 0)),
        ],
        out_specs=pl.BlockSpec((N // 10, F), lambda i: (i, 0)),
        out_shape=jax.ShapeDtypeStruct((N, F), jnp.float32),
    )(h, W, b)


def _mm_fused(p, W, b, *, relu):
    def body(p_ref, w_ref, b_ref, o_ref):
        h_blk = p_ref[0] + p_ref[1]
        if relu:
            h_blk = jnp.maximum(h_blk, 0.0)
        o_ref[...] = (
            jnp.dot(h_blk, w_ref[...], preferred_element_type=jnp.float32)
            + b_ref[...]
        )

    return pl.pallas_call(
        body,
        grid=(10,),
        in_specs=[
            pl.BlockSpec((2, N // 10, F), lambda i: (0, i, 0)),
            pl.BlockSpec((F, F), lambda i: (0, 0)),
            pl.BlockSpec((1, F), lambda i: (0, 0)),
        ],
        out_specs=pl.BlockSpec((N // 10, F), lambda i: (i, 0)),
        out_shape=jax.ShapeDtypeStruct((N, F), jnp.float32),
    )(p, W, b)


def _add2(p):
    def body(p_ref, o_ref):
        o_ref[...] = p_ref[0] + p_ref[1]

    return pl.pallas_call(
        body,
        grid=(10,),
        in_specs=[pl.BlockSpec((2, N // 10, F), lambda i: (0, i, 0))],
        out_specs=pl.BlockSpec((N // 10, F), lambda i: (i, 0)),
        out_shape=jax.ShapeDtypeStruct((N, F), jnp.float32),
    )(p)


_MESH = plsc.VectorSubcoreMesh(
    core_axis_name="c", subcore_axis_name="s", num_cores=NC, num_subcores=NS
)


@functools.partial(
    pl.kernel,
    out_type=jax.ShapeDtypeStruct((2, N, F), jnp.float32),
    mesh=_MESH,
    scratch_types=[
        pltpu.VMEM((CHP, K), jnp.int32),
        pltpu.VMEM((CHP, K), jnp.int32),
        pltpu.VMEM((CHP, K), jnp.float32),
        pltpu.VMEM((2, K, F), jnp.float32),
        pltpu.VMEM_SHARED((N, F), jnp.float32),
        pltpu.SemaphoreType.DMA((2,)),
    ],
)
def _prop_kernel(u_hbm, src_hbm, dst_hbm, w_hbm, out_hbm,
                 src_v, dst_v, w_v, buf, acc, gsems):
    c = lax.axis_index("c")
    s = lax.axis_index("s")
    wid = s * NC + c

    zero = jnp.zeros((LANES,), jnp.float32)

    def zrow(i, carry):
        for q in range(F // LANES):
            buf[0, i, pl.ds(q * LANES, LANES)] = zero
        return carry

    lax.fori_loop(0, K, zrow, 0)
    for off in range(0, ROWS, K):
        zn = min(K, ROWS - off)
        pltpu.sync_copy(buf.at[0, pl.ds(0, zn)],
                        acc.at[pl.ds(s * ROWS + off, zn)])

    @pl.when(s == NS - 1)
    def _zero_tail():
        pltpu.sync_copy(buf.at[0, pl.ds(0, TAIL)],
                        acc.at[pl.ds(NS * ROWS, TAIL)])

    plsc.subcore_barrier()

    def scale(bufb, j):
        def group(g, carry2):
            w16 = w_v[j, pl.ds(g * LANES, LANES)]
            base = g * LANES
            for t in range(LANES):
                wsp = lax.gather(
                    w16, jnp.full((LANES, 1), t, jnp.int32),
                    lax.GatherDimensionNumbers(offset_dims=(),
                                               collapsed_slice_dims=(0,),
                                               start_index_map=(0,)),
                    (1,), mode=lax.GatherScatterMode.PROMISE_IN_BOUNDS)
                for q in range(F // LANES):
                    sl = pl.ds(q * LANES, LANES)
                    bufb[base + t, sl] = bufb[base + t, sl] * wsp
            return carry2

        lax.fori_loop(0, K // LANES, group, 0)

    for p in range(PASSES):
        pltpu.sync_copy(src_hbm.at[wid, p], src_v)
        pltpu.sync_copy(dst_hbm.at[wid, p], dst_v)
        pltpu.sync_copy(w_hbm.at[wid, p], w_v)
        pltpu.async_copy(u_hbm.at[src_v.at[0]], buf.at[0], gsems.at[0])

        def step_fn(step, carry):
            for b in range(2):
                j = 2 * step + b
                bn = 1 - b
                jn = jnp.minimum(j + 1, CHP - 1)
                pltpu.async_copy(u_hbm.at[src_v.at[jn]], buf.at[bn],
                                 gsems.at[bn])
                pltpu.make_async_copy(u_hbm.at[src_v.at[j]], buf.at[b],
                                      gsems.at[b]).wait()
                scale(buf.at[b], j)
                pltpu.sync_copy(buf.at[b], acc.at[dst_v.at[j]], add=True)
            return carry

        lax.fori_loop(0, CHP // 2, step_fn, 0)
        pltpu.make_async_copy(u_hbm.at[src_v.at[CHP - 1]], buf.at[0],
                              gsems.at[0]).wait()

    plsc.subcore_barrier()
    pltpu.sync_copy(acc.at[pl.ds(s * ROWS, ROWS)],
                    out_hbm.at[c, pl.ds(s * ROWS, ROWS)])

    @pl.when(s == NS - 1)
    def _copy_tail():
        pltpu.sync_copy(acc.at[pl.ds(NS * ROWS, TAIL)],
                        out_hbm.at[c, pl.ds(NS * ROWS, TAIL)])


def kernel(x, edge_index, edge_weight, W1, b1, W2, b2, W3, b3):
    pad = ((0, 0), (0, EPWP - EPW))
    src = jnp.pad(edge_index[0].reshape(NW, EPW), pad).reshape(NW, PASSES, CHP, K)
    dst = jnp.pad(edge_index[1].reshape(NW, EPW), pad).reshape(NW, PASSES, CHP, K)
    w = jnp.pad(edge_weight.reshape(NW, EPW), pad).reshape(NW, PASSES, CHP, K)

    u = _mm(x, W1, b1.reshape(1, F))
    p = _prop_kernel(u, src, dst, w)
    u = _mm_fused(p, W2, b2.reshape(1, F), relu=True)
    p = _prop_kernel(u, src, dst, w)
    u = _mm_fused(p, W3, b3.reshape(1, F), relu=True)
    p = _prop_kernel(u, src, dst, w)
    return _add2(p)


# speedup vs baseline: 1.7015x; 1.1336x over previous
import functools

import jax
import jax.numpy as jnp
from jax import lax
from jax.experimental import pallas as pl
from jax.experimental.pallas import tpu as pltpu
from jax.experimental.pallas import tpu_sc as plsc

N = 10000
E = 320000
F = 128

NC = 2
NS = 16
NW = NC * NS
EPW = E // NW
K = 128
EPWP = 10240
CH = EPWP // K
ROWS = 624
TAIL = N - NS * ROWS
LANES = 16


def _mm(h, W, b, *, relu=False):
    def body(h_ref, w_ref, b_ref, o_ref):
        h_blk = h_ref[...]
        if relu:
            h_blk = jnp.maximum(h_blk, 0.0)
        o_ref[...] = (
            jnp.dot(h_blk, w_ref[...], preferred_element_type=jnp.float32)
            + b_ref[...]
        )

    return pl.pallas_call(
        body,
        grid=(10,),
        in_specs=[
            pl.BlockSpec((N // 10, F), lambda i: (i, 0)),
            pl.BlockSpec((F, F), lambda i: (0, 0)),
            pl.BlockSpec((1, F), lambda i: (0, 0)),
        ],
        out_specs=pl.BlockSpec((N // 10, F), lambda i: (i, 0)),
        out_shape=jax.ShapeDtypeStruct((N, F), jnp.float32),
    )(h, W, b)


def _mm_fused(p, W, b, *, relu):
    def body(p_ref, w_ref, b_ref, o_ref):
        h_blk = p_ref[0] + p_ref[1]
        if relu:
            h_blk = jnp.maximum(h_blk, 0.0)
        o_ref[...] = (
            jnp.dot(h_blk, w_ref[...], preferred_element_type=jnp.float32)
            + b_ref[...]
        )

    return pl.pallas_call(
        body,
        grid=(10,),
        in_specs=[
            pl.BlockSpec((2, N // 10, F), lambda i: (0, i, 0)),
            pl.BlockSpec((F, F), lambda i: (0, 0)),
            pl.BlockSpec((1, F), lambda i: (0, 0)),
        ],
        out_specs=pl.BlockSpec((N // 10, F), lambda i: (i, 0)),
        out_shape=jax.ShapeDtypeStruct((N, F), jnp.float32),
    )(p, W, b)


def _add2(p):
    def body(p_ref, o_ref):
        o_ref[...] = p_ref[0] + p_ref[1]

    return pl.pallas_call(
        body,
        grid=(10,),
        in_specs=[pl.BlockSpec((2, N // 10, F), lambda i: (0, i, 0))],
        out_specs=pl.BlockSpec((N // 10, F), lambda i: (i, 0)),
        out_shape=jax.ShapeDtypeStruct((N, F), jnp.float32),
    )(p)


_MESH = plsc.VectorSubcoreMesh(
    core_axis_name="c", subcore_axis_name="s", num_cores=NC, num_subcores=NS
)


@functools.partial(
    pl.kernel,
    out_type=jax.ShapeDtypeStruct((2, N, F), jnp.float32),
    mesh=_MESH,
    scratch_types=[
        pltpu.VMEM((CH, K), jnp.int32),
        pltpu.VMEM((CH, K), jnp.int32),
        pltpu.VMEM((CH, K), jnp.float32),
        pltpu.VMEM((K, F), jnp.float32),
        pltpu.VMEM_SHARED((N, F), jnp.float32),
        pltpu.SemaphoreType.DMA,
    ],
)
def _prop_kernel(u_hbm, src_hbm, dst_hbm, w_hbm, out_hbm,
                 src_v, dst_v, w_v, buf, acc, gsem):
    c = lax.axis_index("c")
    s = lax.axis_index("s")
    wid = s * NC + c

    zero = jnp.zeros((LANES,), jnp.float32)

    def zrow(i, carry):
        for q in range(F // LANES):
            buf[i, pl.ds(q * LANES, LANES)] = zero
        return carry

    lax.fori_loop(0, K, zrow, 0)
    for off in range(0, ROWS, K):
        zn = min(K, ROWS - off)
        pltpu.sync_copy(buf.at[pl.ds(0, zn)],
                        acc.at[pl.ds(s * ROWS + off, zn)])

    @pl.when(s == NS - 1)
    def _zero_tail():
        pltpu.sync_copy(buf.at[pl.ds(0, TAIL)],
                        acc.at[pl.ds(NS * ROWS, TAIL)])

    plsc.subcore_barrier()

    def scale(bufb, j):
        def group(g, carry2):
            w16 = w_v[j, pl.ds(g * LANES, LANES)]
            base = g * LANES
            for t in range(LANES):
                wsp = lax.gather(
                    w16, jnp.full((LANES, 1), t, jnp.int32),
                    lax.GatherDimensionNumbers(offset_dims=(),
                                               collapsed_slice_dims=(0,),
                                               start_index_map=(0,)),
                    (1,), mode=lax.GatherScatterMode.PROMISE_IN_BOUNDS)
                for q in range(F // LANES):
                    sl = pl.ds(q * LANES, LANES)
                    bufb[base + t, sl] = bufb[base + t, sl] * wsp
            return carry2

        lax.fori_loop(0, K // LANES, group, 0)

    pltpu.sync_copy(src_hbm.at[wid], src_v)
    pltpu.sync_copy(dst_hbm.at[wid], dst_v)
    pltpu.sync_copy(w_hbm.at[wid], w_v)

    def chunk(j, carry):
        pltpu.async_copy(u_hbm.at[src_v.at[j]], buf, gsem).wait()
        scale(buf, j)
        pltpu.sync_copy(buf, acc.at[dst_v.at[j]], add=True)
        return carry

    lax.fori_loop(0, CH, chunk, 0)

    plsc.subcore_barrier()
    pltpu.sync_copy(acc.at[pl.ds(s * ROWS, ROWS)],
                    out_hbm.at[c, pl.ds(s * ROWS, ROWS)])

    @pl.when(s == NS - 1)
    def _copy_tail():
        pltpu.sync_copy(acc.at[pl.ds(NS * ROWS, TAIL)],
                        out_hbm.at[c, pl.ds(NS * ROWS, TAIL)])


def kernel(x, edge_index, edge_weight, W1, b1, W2, b2, W3, b3):
    pad = ((0, 0), (0, EPWP - EPW))
    src = jnp.pad(edge_index[0].reshape(NW, EPW), pad).reshape(NW, CH, K)
    dst = jnp.pad(edge_index[1].reshape(NW, EPW), pad).reshape(NW, CH, K)
    w = jnp.pad(edge_weight.reshape(NW, EPW), pad).reshape(NW, CH, K)

    u = _mm(x, W1, b1.reshape(1, F))
    p = _prop_kernel(u, src, dst, w)
    u = _mm_fused(p, W2, b2.reshape(1, F), relu=True)
    p = _prop_kernel(u, src, dst, w)
    u = _mm_fused(p, W3, b3.reshape(1, F), relu=True)
    p = _prop_kernel(u, src, dst, w)
    return _add2(p)


# speedup vs baseline: 1.7266x; 1.0148x over previous
import functools

import jax
import jax.numpy as jnp
from jax import lax
from jax.experimental import pallas as pl
from jax.experimental.pallas import tpu as pltpu
from jax.experimental.pallas import tpu_sc as plsc

N = 10000
E = 320000
F = 128

NC = 2
NS = 16
NW = NC * NS
EPW = E // NW
K = 128
EPWP = 10240
CH = EPWP // K
ROWS = 624
TAIL = N - NS * ROWS
LANES = 16


def _mm(h, W, b, *, relu=False):
    def body(h_ref, w_ref, b_ref, o_ref):
        h_blk = h_ref[...]
        if relu:
            h_blk = jnp.maximum(h_blk, 0.0)
        o_ref[...] = (
            jnp.dot(h_blk, w_ref[...], preferred_element_type=jnp.float32)
            + b_ref[...]
        )

    return pl.pallas_call(
        body,
        grid=(10,),
        in_specs=[
            pl.BlockSpec((N // 10, F), lambda i: (i, 0)),
            pl.BlockSpec((F, F), lambda i: (0, 0)),
            pl.BlockSpec((1, F), lambda i: (0, 0)),
        ],
        out_specs=pl.BlockSpec((N // 10, F), lambda i: (i, 0)),
        out_shape=jax.ShapeDtypeStruct((N, F), jnp.float32),
    )(h, W, b)


def _mm_fused(p, W, b, *, relu):
    def body(p_ref, w_ref, b_ref, o_ref):
        h_blk = p_ref[0] + p_ref[1]
        if relu:
            h_blk = jnp.maximum(h_blk, 0.0)
        o_ref[...] = (
            jnp.dot(h_blk, w_ref[...], preferred_element_type=jnp.float32)
            + b_ref[...]
        )

    return pl.pallas_call(
        body,
        grid=(10,),
        in_specs=[
            pl.BlockSpec((2, N // 10, F), lambda i: (0, i, 0)),
            pl.BlockSpec((F, F), lambda i: (0, 0)),
            pl.BlockSpec((1, F), lambda i: (0, 0)),
        ],
        out_specs=pl.BlockSpec((N // 10, F), lambda i: (i, 0)),
        out_shape=jax.ShapeDtypeStruct((N, F), jnp.float32),
    )(p, W, b)


def _add2(p):
    def body(p_ref, o_ref):
        o_ref[...] = p_ref[0] + p_ref[1]

    return pl.pallas_call(
        body,
        grid=(10,),
        in_specs=[pl.BlockSpec((2, N // 10, F), lambda i: (0, i, 0))],
        out_specs=pl.BlockSpec((N // 10, F), lambda i: (i, 0)),
        out_shape=jax.ShapeDtypeStruct((N, F), jnp.float32),
    )(p)


_MESH = plsc.VectorSubcoreMesh(
    core_axis_name="c", subcore_axis_name="s", num_cores=NC, num_subcores=NS
)


@functools.partial(
    pl.kernel,
    out_type=jax.ShapeDtypeStruct((2, N, F), jnp.float32),
    mesh=_MESH,
    scratch_types=[
        pltpu.VMEM((CH, K), jnp.int32),
        pltpu.VMEM((CH, K), jnp.int32),
        pltpu.VMEM((CH, K), jnp.float32),
        pltpu.VMEM((K, F), jnp.float32),
        pltpu.VMEM_SHARED((N, F), jnp.float32),
        pltpu.SemaphoreType.DMA((2,)),
    ],
)
def _prop_kernel(u_hbm, src_hbm, dst_hbm, w_hbm, out_hbm,
                 src_v, dst_v, w_v, buf, acc, gsem):
    c = lax.axis_index("c")
    s = lax.axis_index("s")
    wid = s * NC + c

    zero = jnp.zeros((LANES,), jnp.float32)

    def zrow(i, carry):
        for q in range(F // LANES):
            buf[i, pl.ds(q * LANES, LANES)] = zero
        return carry

    lax.fori_loop(0, K, zrow, 0)
    for off in range(0, ROWS, K):
        zn = min(K, ROWS - off)
        pltpu.sync_copy(buf.at[pl.ds(0, zn)],
                        acc.at[pl.ds(s * ROWS + off, zn)])

    @pl.when(s == NS - 1)
    def _zero_tail():
        pltpu.sync_copy(buf.at[pl.ds(0, TAIL)],
                        acc.at[pl.ds(NS * ROWS, TAIL)])

    plsc.subcore_barrier()

    def scale(bufb, j, g0):
        def group(g, carry2):
            w16 = w_v[j, pl.ds(g * LANES, LANES)]
            base = g * LANES
            for t in range(LANES):
                wsp = lax.gather(
                    w16, jnp.full((LANES, 1), t, jnp.int32),
                    lax.GatherDimensionNumbers(offset_dims=(),
                                               collapsed_slice_dims=(0,),
                                               start_index_map=(0,)),
                    (1,), mode=lax.GatherScatterMode.PROMISE_IN_BOUNDS)
                for q in range(F // LANES):
                    sl = pl.ds(q * LANES, LANES)
                    bufb[base + t, sl] = bufb[base + t, sl] * wsp
            return carry2

        lax.fori_loop(g0, g0 + K // (2 * LANES), group, 0)

    pltpu.sync_copy(src_hbm.at[wid], src_v)
    pltpu.sync_copy(dst_hbm.at[wid], dst_v)
    pltpu.sync_copy(w_hbm.at[wid], w_v)

    H = K // 2

    def chunk(j, carry):
        d1 = pltpu.async_copy(u_hbm.at[src_v.at[j, pl.ds(0, H)]],
                              buf.at[pl.ds(0, H)], gsem.at[0])
        d2 = pltpu.async_copy(u_hbm.at[src_v.at[j, pl.ds(H, H)]],
                              buf.at[pl.ds(H, H)], gsem.at[1])
        d1.wait()
        scale(buf, j, 0)
        d2.wait()
        scale(buf, j, K // (2 * LANES))
        pltpu.sync_copy(buf, acc.at[dst_v.at[j]], add=True)
        return carry

    lax.fori_loop(0, CH, chunk, 0)

    plsc.subcore_barrier()
    pltpu.sync_copy(acc.at[pl.ds(s * ROWS, ROWS)],
                    out_hbm.at[c, pl.ds(s * ROWS, ROWS)])

    @pl.when(s == NS - 1)
    def _copy_tail():
        pltpu.sync_copy(acc.at[pl.ds(NS * ROWS, TAIL)],
                        out_hbm.at[c, pl.ds(NS * ROWS, TAIL)])


def kernel(x, edge_index, edge_weight, W1, b1, W2, b2, W3, b3):
    pad = ((0, 0), (0, EPWP - EPW))
    src = jnp.pad(edge_index[0].reshape(NW, EPW), pad).reshape(NW, CH, K)
    dst = jnp.pad(edge_index[1].reshape(NW, EPW), pad).reshape(NW, CH, K)
    w = jnp.pad(edge_weight.reshape(NW, EPW), pad).reshape(NW, CH, K)

    u = _mm(x, W1, b1.reshape(1, F))
    p = _prop_kernel(u, src, dst, w)
    u = _mm_fused(p, W2, b2.reshape(1, F), relu=True)
    p = _prop_kernel(u, src, dst, w)
    u = _mm_fused(p, W3, b3.reshape(1, F), relu=True)
    p = _prop_kernel(u, src, dst, w)
    return _add2(p)


# speedup vs baseline: 1.7337x; 1.0041x over previous
import functools

import jax
import jax.numpy as jnp
from jax import lax
from jax.experimental import pallas as pl
from jax.experimental.pallas import tpu as pltpu
from jax.experimental.pallas import tpu_sc as plsc

N = 10000
E = 320000
F = 128

NC = 2
NS = 16
NW = NC * NS
EPW = E // NW
K = 128
EPWP = 10240
CH = EPWP // K
ROWS = 624
TAIL = N - NS * ROWS
LANES = 16


def _mm(h, W, b, *, relu=False):
    def body(h_ref, w_ref, b_ref, o_ref):
        h_blk = h_ref[...]
        if relu:
            h_blk = jnp.maximum(h_blk, 0.0)
        o_ref[...] = (
            jnp.dot(h_blk, w_ref[...], preferred_element_type=jnp.float32)
            + b_ref[...]
        )

    return pl.pallas_call(
        body,
        grid=(10,),
        in_specs=[
            pl.BlockSpec((N // 10, F), lambda i: (i, 0)),
            pl.BlockSpec((F, F), lambda i: (0, 0)),
            pl.BlockSpec((1, F), lambda i: (0, 0)),
        ],
        out_specs=pl.BlockSpec((N // 10, F), lambda i: (i, 0)),
        out_shape=jax.ShapeDtypeStruct((N, F), jnp.float32),
    )(h, W, b)


def _mm_fused(p, W, b, *, relu):
    def body(p_ref, w_ref, b_ref, o_ref):
        h_blk = p_ref[0] + p_ref[1]
        if relu:
            h_blk = jnp.maximum(h_blk, 0.0)
        o_ref[...] = (
            jnp.dot(h_blk, w_ref[...], preferred_element_type=jnp.float32)
            + b_ref[...]
        )

    return pl.pallas_call(
        body,
        grid=(10,),
        in_specs=[
            pl.BlockSpec((2, N // 10, F), lambda i: (0, i, 0)),
            pl.BlockSpec((F, F), lambda i: (0, 0)),
            pl.BlockSpec((1, F), lambda i: (0, 0)),
        ],
        out_specs=pl.BlockSpec((N // 10, F), lambda i: (i, 0)),
        out_shape=jax.ShapeDtypeStruct((N, F), jnp.float32),
    )(p, W, b)


def _add2(p):
    def body(p_ref, o_ref):
        o_ref[...] = p_ref[0] + p_ref[1]

    return pl.pallas_call(
        body,
        grid=(10,),
        in_specs=[pl.BlockSpec((2, N // 10, F), lambda i: (0, i, 0))],
        out_specs=pl.BlockSpec((N // 10, F), lambda i: (i, 0)),
        out_shape=jax.ShapeDtypeStruct((N, F), jnp.float32),
    )(p)


_MESH = plsc.VectorSubcoreMesh(
    core_axis_name="c", subcore_axis_name="s", num_cores=NC, num_subcores=NS
)


@functools.partial(
    pl.kernel,
    out_type=jax.ShapeDtypeStruct((2, N, F), jnp.float32),
    mesh=_MESH,
    scratch_types=[
        pltpu.VMEM((CH, K), jnp.int32),
        pltpu.VMEM((CH, K), jnp.int32),
        pltpu.VMEM((CH, K), jnp.float32),
        pltpu.VMEM((K, F), jnp.float32),
        pltpu.VMEM_SHARED((N, F), jnp.float32),
        pltpu.SemaphoreType.DMA((4,)),
    ],
)
def _prop_kernel(u_hbm, src_hbm, dst_hbm, w_hbm, out_hbm,
                 src_v, dst_v, w_v, buf, acc, gsem):
    c = lax.axis_index("c")
    s = lax.axis_index("s")
    wid = s * NC + c

    zero = jnp.zeros((LANES,), jnp.float32)

    def zrow(i, carry):
        for q in range(F // LANES):
            buf[i, pl.ds(q * LANES, LANES)] = zero
        return carry

    lax.fori_loop(0, K, zrow, 0)
    for off in range(0, ROWS, K):
        zn = min(K, ROWS - off)
        pltpu.sync_copy(buf.at[pl.ds(0, zn)],
                        acc.at[pl.ds(s * ROWS + off, zn)])

    @pl.when(s == NS - 1)
    def _zero_tail():
        pltpu.sync_copy(buf.at[pl.ds(0, TAIL)],
                        acc.at[pl.ds(NS * ROWS, TAIL)])

    plsc.subcore_barrier()

    def scale(bufb, j, g0):
        def group(g, carry2):
            w16 = w_v[j, pl.ds(g * LANES, LANES)]
            base = g * LANES
            for t in range(LANES):
                wsp = lax.gather(
                    w16, jnp.full((LANES, 1), t, jnp.int32),
                    lax.GatherDimensionNumbers(offset_dims=(),
                                               collapsed_slice_dims=(0,),
                                               start_index_map=(0,)),
                    (1,), mode=lax.GatherScatterMode.PROMISE_IN_BOUNDS)
                for q in range(F // LANES):
                    sl = pl.ds(q * LANES, LANES)
                    bufb[base + t, sl] = bufb[base + t, sl] * wsp
            return carry2

        lax.fori_loop(g0, g0 + K // (4 * LANES), group, 0)

    pltpu.sync_copy(src_hbm.at[wid], src_v)
    pltpu.sync_copy(dst_hbm.at[wid], dst_v)
    pltpu.sync_copy(w_hbm.at[wid], w_v)

    Q = K // 4

    def chunk(j, carry):
        ds = [pltpu.async_copy(u_hbm.at[src_v.at[j, pl.ds(q * Q, Q)]],
                               buf.at[pl.ds(q * Q, Q)], gsem.at[q])
              for q in range(4)]
        for q in range(4):
            ds[q].wait()
            scale(buf, j, q * (Q // LANES))
        pltpu.sync_copy(buf, acc.at[dst_v.at[j]], add=True)
        return carry

    lax.fori_loop(0, CH, chunk, 0)

    plsc.subcore_barrier()
    pltpu.sync_copy(acc.at[pl.ds(s * ROWS, ROWS)],
                    out_hbm.at[c, pl.ds(s * ROWS, ROWS)])

    @pl.when(s == NS - 1)
    def _copy_tail():
        pltpu.sync_copy(acc.at[pl.ds(NS * ROWS, TAIL)],
                        out_hbm.at[c, pl.ds(NS * ROWS, TAIL)])


def kernel(x, edge_index, edge_weight, W1, b1, W2, b2, W3, b3):
    pad = ((0, 0), (0, EPWP - EPW))
    src = jnp.pad(edge_index[0].reshape(NW, EPW), pad).reshape(NW, CH, K)
    dst = jnp.pad(edge_index[1].reshape(NW, EPW), pad).reshape(NW, CH, K)
    w = jnp.pad(edge_weight.reshape(NW, EPW), pad).reshape(NW, CH, K)

    u = _mm(x, W1, b1.reshape(1, F))
    p = _prop_kernel(u, src, dst, w)
    u = _mm_fused(p, W2, b2.reshape(1, F), relu=True)
    p = _prop_kernel(u, src, dst, w)
    u = _mm_fused(p, W3, b3.reshape(1, F), relu=True)
    p = _prop_kernel(u, src, dst, w)
    return _add2(p)
